# baseline (speedup 1.0000x reference)
import functools

import jax
import jax.numpy as jnp
from jax.experimental import pallas as pl
from jax.experimental.pallas import tpu as pltpu

_BB = 256
_KB = 2048
_NBUF = 8
_NPRI = 2


def _mv_kernel(x_hbm, w_ref, b_ref, o_ref, bufs, tailbuf, sems, tail_sem,
               *, nk, tail):
    i = pl.program_id(0)
    row = i * _BB

    def chunk_copy(k, slot):
        return pltpu.make_async_copy(
            x_hbm.at[pl.ds(row, _BB), pl.ds(k * _KB, _KB)],
            bufs.at[slot],
            sems.at[slot],
        )

    tail_copy = pltpu.make_async_copy(
        x_hbm.at[pl.ds(row, _BB), pl.ds(nk * _KB, tail)],
        tailbuf,
        tail_sem,
    )
    tail_copy.start()
    for k in range(_NBUF):
        chunk_copy(k, k).start(priority=k % _NPRI)

    o_ref[...] = jnp.broadcast_to(b_ref[0, 0], o_ref.shape)

    def body(k, _):
        slot = jax.lax.rem(k, _NBUF)
        chunk_copy(k, slot).wait()
        wc = w_ref[pl.ds(k, 1), :]
        o_ref[...] += jnp.sum(bufs[slot] * wc, axis=1, keepdims=True)

        nxt = k + _NBUF

        for p in range(_NPRI):
            @pl.when(jnp.logical_and(nxt < nk, slot % _NPRI == p))
            def _():
                chunk_copy(nxt, slot).start(priority=p)

        return 0

    jax.lax.fori_loop(0, nk, body, 0)

    tail_copy.wait()
    wt = w_ref[pl.ds(nk, 1), :tail]
    o_ref[...] += jnp.sum(tailbuf[...] * wt, axis=1, keepdims=True)


@jax.jit
def kernel(x, weight, bias):
    batch, vocab = x.shape
    nk = vocab // _KB
    tail = vocab - nk * _KB
    wpad = jnp.pad(weight.reshape(-1), (0, (nk + 1) * _KB - vocab))
    w2 = wpad.reshape(nk + 1, _KB)
    out = pl.pallas_call(
        functools.partial(_mv_kernel, nk=nk, tail=tail),
        grid=(batch // _BB,),
        in_specs=[
            pl.BlockSpec(memory_space=pltpu.MemorySpace.HBM),
            pl.BlockSpec((nk + 1, _KB), lambda i: (0, 0)),
            pl.BlockSpec((1, 1), lambda i: (0, 0)),
        ],
        out_specs=pl.BlockSpec((_BB, 1), lambda i: (i, 0)),
        out_shape=jax.ShapeDtypeStruct((batch, 1), jnp.float32),
        scratch_shapes=[
            pltpu.VMEM((_NBUF, _BB, _KB), jnp.float32),
            pltpu.VMEM((_BB, tail), jnp.float32),
            pltpu.SemaphoreType.DMA((_NBUF,)),
            pltpu.SemaphoreType.DMA,
        ],
        compiler_params=pltpu.CompilerParams(
            dimension_semantics=("parallel",)
        ),
    )(x, w2, bias.reshape(1, 1))
    return out
